# Initial kernel scaffold; baseline (speedup 1.0000x reference)
import jax
import jax.numpy as jnp
from jax.experimental import pallas as pl


def kernel(node_feats, hyperedge_index, num_hyperedges, Wk, Wv):
    raise NotImplementedError("write your pallas kernel here")


# speedup vs baseline: 2.3321x; 2.3321x over previous
import jax
import jax.numpy as jnp
from jax import lax
from jax.experimental import pallas as pl
from jax.experimental.pallas import tpu as pltpu
from jax.experimental.pallas import tpu_sc as plsc

N = 10000
E = 160000
D = 256
P = 64
H = 5000

NC = 2
NS = 16
NW = NC * NS
LANES = 16

W = 128
HP = 5120
ROWS_PER_TILE = HP // NS
EP = 163840
EDGES_PER_TILE = EP // NW
C = 128
NCHUNK = EDGES_PER_TILE // C

_SCALE = 1.0 / (P ** 0.5)


def _proj_body(x_ref, wk_ref, wv_ref, katab_ref, kvtab_ref):
    x = x_ref[...]
    dn = (((1,), (1,)), ((), ()))
    k = lax.dot_general(x, wk_ref[...], dn, preferred_element_type=jnp.float32)
    v = lax.dot_general(x, wv_ref[...], dn, preferred_element_type=jnp.float32)
    ones = jnp.ones((N, 1), jnp.float32)
    zeros = jnp.zeros((N, W - P - 1), jnp.float32)
    katab_ref[...] = jnp.concatenate([k, ones, zeros], axis=1)
    kvtab_ref[...] = jnp.concatenate([k, v], axis=1)


def _mid_body(sumk_ref, ctab_ref):
    sumk = sumk_ref[0, :, :P] + sumk_ref[1, :, :P]
    cnt = sumk_ref[0, :, P] + sumk_ref[1, :, P]
    cen = sumk / jnp.maximum(cnt, 1.0)[:, None]
    ctab_ref[...] = jnp.concatenate(
        [cen, jnp.zeros((HP, W - P), jnp.float32)], axis=1)


def _final_body(acc_ref, wv_ref, out_ref):
    a = acc_ref[0, :H, :P] + acc_ref[1, :H, :P]
    den = acc_ref[0, :H, P] + acc_ref[1, :H, P]
    agg = jnp.where((den > 0.0)[:, None], a / den[:, None], 0.0)
    out_ref[...] = jnp.dot(agg, wv_ref[...], preferred_element_type=jnp.float32)


def _chunked_rows(fn):
    off = 0
    while off < ROWS_PER_TILE:
        n = min(C, ROWS_PER_TILE - off)
        fn(off, n)
        off += n


def _pass_a_body(nid_hbm, he_hbm, katab_hbm, ztab_hbm,
                 sumk_out,
                 nid_v, he_v, rows_v, sumk_s):
    c = lax.axis_index("c")
    s = lax.axis_index("s")
    wid = c * NS + s
    rbase = s * ROWS_PER_TILE

    def zero_sumk(o, n):
        pltpu.sync_copy(ztab_hbm.at[pl.ds(rbase + o, n)], rows_v.at[pl.ds(0, n)])
        pltpu.sync_copy(rows_v.at[pl.ds(0, n)], sumk_s.at[pl.ds(rbase + o, n)])
    _chunked_rows(zero_sumk)
    plsc.subcore_barrier()

    ebase = wid * EDGES_PER_TILE

    def chunk_body(i, carry):
        off = ebase + i * C
        pltpu.sync_copy(nid_hbm.at[pl.ds(off, C)], nid_v)
        pltpu.sync_copy(he_hbm.at[pl.ds(off, C)], he_v)
        pltpu.sync_copy(katab_hbm.at[nid_v], rows_v)
        pltpu.sync_copy(rows_v, sumk_s.at[he_v], add=True)
        return carry
    lax.fori_loop(0, NCHUNK, chunk_body, 0)
    plsc.subcore_barrier()

    def out_sumk(o, n):
        pltpu.sync_copy(sumk_s.at[pl.ds(rbase + o, n)], rows_v.at[pl.ds(0, n)])
        pltpu.sync_copy(rows_v.at[pl.ds(0, n)],
                        sumk_out.at[c, pl.ds(rbase + o, n)])
    _chunked_rows(out_sumk)


def _pass_b_body(nid_hbm, he_hbm, kvtab_hbm, ctab_hbm, ztab_hbm,
                 acc_out,
                 nid_v, he_v, kv_v, c_v, wv_v, acc_s):
    c = lax.axis_index("c")
    s = lax.axis_index("s")
    wid = c * NS + s
    rbase = s * ROWS_PER_TILE

    def zero_acc(o, n):
        pltpu.sync_copy(ztab_hbm.at[pl.ds(rbase + o, n)], wv_v.at[pl.ds(0, n)])
        pltpu.sync_copy(wv_v.at[pl.ds(0, n)], acc_s.at[pl.ds(rbase + o, n)])
    _chunked_rows(zero_acc)
    plsc.subcore_barrier()

    lane = lax.iota(jnp.int32, LANES)
    ebase = wid * EDGES_PER_TILE

    def egroup_body(g, carry):
        idx0 = g * LANES + lane
        d2 = jnp.zeros((LANES,), jnp.float32)
        for p in range(P):
            pc = jnp.full((LANES,), p, jnp.int32)
            kcol = plsc.load_gather(kv_v, [idx0, pc])
            ccol = plsc.load_gather(c_v, [idx0, pc])
            d = kcol - ccol
            d2 = d2 + d * d
        x = jnp.maximum(d2, 1e-24)
        seed = lax.shift_right_logical(plsc.bitcast(x, jnp.int32), 1) + 0x1FBD1DF5
        y = plsc.bitcast(seed, jnp.float32)
        y = 0.5 * (y + x / y)
        y = 0.5 * (y + x / y)
        y = 0.5 * (y + x / y)
        ex = jnp.exp(y * _SCALE)
        plsc.store_scatter(wv_v, [idx0, jnp.full((LANES,), P, jnp.int32)], ex)
        for p in range(P):
            vcol = plsc.load_gather(kv_v, [idx0, jnp.full((LANES,), P + p, jnp.int32)])
            plsc.store_scatter(wv_v, [idx0, jnp.full((LANES,), p, jnp.int32)], vcol * ex)
        return carry

    def chunk_body(i, carry):
        off = ebase + i * C
        pltpu.sync_copy(nid_hbm.at[pl.ds(off, C)], nid_v)
        pltpu.sync_copy(he_hbm.at[pl.ds(off, C)], he_v)
        pltpu.sync_copy(kvtab_hbm.at[nid_v], kv_v)
        pltpu.sync_copy(ctab_hbm.at[he_v], c_v)
        lax.fori_loop(0, C // LANES, egroup_body, 0)
        pltpu.sync_copy(wv_v, acc_s.at[he_v], add=True)
        return carry
    lax.fori_loop(0, NCHUNK, chunk_body, 0)
    plsc.subcore_barrier()

    def out_acc(o, n):
        pltpu.sync_copy(acc_s.at[pl.ds(rbase + o, n)], wv_v.at[pl.ds(0, n)])
        pltpu.sync_copy(wv_v.at[pl.ds(0, n)], acc_out.at[c, pl.ds(rbase + o, n)])
    _chunked_rows(out_acc)


def kernel(node_feats, hyperedge_index, num_hyperedges, Wk, Wv):
    f32 = jnp.float32
    i32 = jnp.int32

    shift = jnp.asarray(num_hyperedges - H, i32)
    nid = hyperedge_index[0]
    he = hyperedge_index[1] + shift
    pad = EP - E
    nid_p = jnp.concatenate([nid, jnp.zeros((pad,), i32)])
    he_p = jnp.concatenate([he, jnp.full((pad,), H, i32)])
    ztab = jnp.zeros((HP, W), f32)

    katab, kvtab = pl.pallas_call(
        _proj_body,
        out_shape=[jax.ShapeDtypeStruct((N, W), f32),
                   jax.ShapeDtypeStruct((N, W), f32)],
    )(node_feats, Wk, Wv)

    mesh = plsc.VectorSubcoreMesh(core_axis_name="c", subcore_axis_name="s",
                                  num_cores=NC, num_subcores=NS)

    pass_a = pl.kernel(
        _pass_a_body,
        out_type=jax.ShapeDtypeStruct((NC, HP, W), f32),
        mesh=mesh,
        scratch_types=[
            pltpu.VMEM((C,), i32),
            pltpu.VMEM((C,), i32),
            pltpu.VMEM((C, W), f32),
            pltpu.VMEM_SHARED((HP, W), f32),
        ],
    )
    sumk_part = pass_a(nid_p, he_p, katab, ztab)

    ctab = pl.pallas_call(
        _mid_body,
        out_shape=jax.ShapeDtypeStruct((HP, W), f32),
    )(sumk_part)

    pass_b = pl.kernel(
        _pass_b_body,
        out_type=jax.ShapeDtypeStruct((NC, HP, W), f32),
        mesh=mesh,
        compiler_params=pltpu.CompilerParams(needs_layout_passes=False),
        scratch_types=[
            pltpu.VMEM((C,), i32),
            pltpu.VMEM((C,), i32),
            pltpu.VMEM((C, W), f32),
            pltpu.VMEM((C, W), f32),
            pltpu.VMEM((C, W), f32),
            pltpu.VMEM_SHARED((HP, W), f32),
        ],
    )
    acc_part = pass_b(nid_p, he_p, kvtab, ctab, ztab)

    out = pl.pallas_call(
        _final_body,
        out_shape=jax.ShapeDtypeStruct((H, D), f32),
    )(acc_part, Wv)
    return out


# speedup vs baseline: 3.3408x; 1.4325x over previous
import jax
import jax.numpy as jnp
from jax import lax
from jax.experimental import pallas as pl
from jax.experimental.pallas import tpu as pltpu
from jax.experimental.pallas import tpu_sc as plsc

N = 10000
E = 160000
D = 256
P = 64
H = 5000

NC = 2
NS = 16
NW = NC * NS
LANES = 16

W = 128
HP = 5120
ROWS_PER_TILE = HP // NS
EP = 163840
EDGES_PER_TILE = EP // NW
C = 128
NCHUNK = EDGES_PER_TILE // C

_SCALE = 1.0 / (P ** 0.5)


def _proj_body(x_ref, wk_ref, wv_ref, katab_ref, kvtab_ref):
    x = x_ref[...]
    dn = (((1,), (1,)), ((), ()))
    k = lax.dot_general(x, wk_ref[...], dn, preferred_element_type=jnp.float32)
    v = lax.dot_general(x, wv_ref[...], dn, preferred_element_type=jnp.float32)
    ones = jnp.ones((N, 1), jnp.float32)
    zeros = jnp.zeros((N, W - P - 1), jnp.float32)
    katab_ref[...] = jnp.concatenate([k, ones, zeros], axis=1)
    kvtab_ref[...] = jnp.concatenate([k, v], axis=1)


def _mid_body(sumk_ref, ctab_ref):
    sumk = sumk_ref[0, :, :P] + sumk_ref[1, :, :P]
    cnt = sumk_ref[0, :, P] + sumk_ref[1, :, P]
    cen = sumk / jnp.maximum(cnt, 1.0)[:, None]
    ctab_ref[...] = jnp.concatenate(
        [cen, jnp.zeros((HP, W - P), jnp.float32)], axis=1)


def _final_body(acc_ref, wv_ref, out_ref):
    a = acc_ref[0, :H, :P] + acc_ref[1, :H, :P]
    den = acc_ref[0, :H, P] + acc_ref[1, :H, P]
    agg = jnp.where((den > 0.0)[:, None], a / den[:, None], 0.0)
    out_ref[...] = jnp.dot(agg, wv_ref[...], preferred_element_type=jnp.float32)


def _chunked_rows(fn):
    off = 0
    while off < ROWS_PER_TILE:
        n = min(C, ROWS_PER_TILE - off)
        fn(off, n)
        off += n


def _pass_a_body(nid_hbm, he_hbm, katab_hbm, ztab_hbm,
                 sumk_out,
                 nid_v0, he_v0, rows_v0, nid_v1, he_v1, rows_v1,
                 sem0, sem1, sumk_s):
    c = lax.axis_index("c")
    s = lax.axis_index("s")
    wid = c * NS + s
    rbase = s * ROWS_PER_TILE

    def zero_sumk(o, n):
        pltpu.sync_copy(ztab_hbm.at[pl.ds(rbase + o, n)], rows_v0.at[pl.ds(0, n)])
        pltpu.sync_copy(rows_v0.at[pl.ds(0, n)], sumk_s.at[pl.ds(rbase + o, n)])
    _chunked_rows(zero_sumk)
    plsc.subcore_barrier()

    ebase = wid * EDGES_PER_TILE
    bufs = ((nid_v0, he_v0, rows_v0, sem0), (nid_v1, he_v1, rows_v1, sem1))

    def start(i, b):
        nid_v, he_v, rows_v, sem = bufs[b]
        off = ebase + i * C
        pltpu.sync_copy(nid_hbm.at[pl.ds(off, C)], nid_v)
        pltpu.sync_copy(he_hbm.at[pl.ds(off, C)], he_v)
        pltpu.async_copy(katab_hbm.at[nid_v], rows_v, sem)

    def wait(b):
        nid_v, he_v, rows_v, sem = bufs[b]
        pltpu.make_async_copy(katab_hbm.at[nid_v], rows_v, sem).wait()

    def scatter(b):
        nid_v, he_v, rows_v, sem = bufs[b]
        pltpu.sync_copy(rows_v, sumk_s.at[he_v], add=True)

    start(0, 0)

    def loop2(j, carry):
        i = j * 2
        start(i + 1, 1)
        wait(0)
        scatter(0)
        nxt = jnp.where(i + 2 < NCHUNK, i + 2, 0)
        start(nxt, 0)
        wait(1)
        scatter(1)
        return carry
    lax.fori_loop(0, NCHUNK // 2, loop2, 0)
    wait(0)
    plsc.subcore_barrier()

    def out_sumk(o, n):
        pltpu.sync_copy(sumk_s.at[pl.ds(rbase + o, n)], rows_v0.at[pl.ds(0, n)])
        pltpu.sync_copy(rows_v0.at[pl.ds(0, n)],
                        sumk_out.at[c, pl.ds(rbase + o, n)])
    _chunked_rows(out_sumk)


def _pass_b_body(nid_hbm, he_hbm, kvtab_hbm, ctab_hbm, ztab_hbm,
                 acc_out,
                 nid_v0, he_v0, kv_v0, c_v0, nid_v1, he_v1, kv_v1, c_v1,
                 wv_v, sem0, sem1, acc_s):
    c = lax.axis_index("c")
    s = lax.axis_index("s")
    wid = c * NS + s
    rbase = s * ROWS_PER_TILE

    def zero_acc(o, n):
        pltpu.sync_copy(ztab_hbm.at[pl.ds(rbase + o, n)], wv_v.at[pl.ds(0, n)])
        pltpu.sync_copy(wv_v.at[pl.ds(0, n)], acc_s.at[pl.ds(rbase + o, n)])
    _chunked_rows(zero_acc)
    plsc.subcore_barrier()

    lane = lax.iota(jnp.int32, LANES)
    ebase = wid * EDGES_PER_TILE
    bufs = ((nid_v0, he_v0, kv_v0, c_v0, sem0), (nid_v1, he_v1, kv_v1, c_v1, sem1))

    def start(i, b):
        nid_v, he_v, kv_v, c_v, sem = bufs[b]
        off = ebase + i * C
        pltpu.sync_copy(nid_hbm.at[pl.ds(off, C)], nid_v)
        pltpu.sync_copy(he_hbm.at[pl.ds(off, C)], he_v)
        pltpu.async_copy(kvtab_hbm.at[nid_v], kv_v, sem)
        pltpu.async_copy(ctab_hbm.at[he_v], c_v, sem)

    def wait(b):
        nid_v, he_v, kv_v, c_v, sem = bufs[b]
        pltpu.make_async_copy(kvtab_hbm.at[nid_v], kv_v, sem).wait()
        pltpu.make_async_copy(ctab_hbm.at[he_v], c_v, sem).wait()

    def compute_scatter(b):
        nid_v, he_v, kv_v, c_v, sem = bufs[b]

        def egroup_body(g, carry):
            idx0 = g * LANES + lane
            d2 = jnp.zeros((LANES,), jnp.float32)
            for p in range(P):
                pc = jnp.full((LANES,), p, jnp.int32)
                kcol = plsc.load_gather(kv_v, [idx0, pc])
                ccol = plsc.load_gather(c_v, [idx0, pc])
                d = kcol - ccol
                d2 = d2 + d * d
            x = jnp.maximum(d2, 1e-24)
            seed = lax.shift_right_logical(plsc.bitcast(x, jnp.int32), 1) + 0x1FBD1DF5
            y = plsc.bitcast(seed, jnp.float32)
            y = 0.5 * (y + x / y)
            y = 0.5 * (y + x / y)
            y = 0.5 * (y + x / y)
            ex = jnp.exp(y * _SCALE)
            plsc.store_scatter(wv_v, [idx0, jnp.full((LANES,), P, jnp.int32)], ex)
            for p in range(P):
                vcol = plsc.load_gather(kv_v, [idx0, jnp.full((LANES,), P + p, jnp.int32)])
                plsc.store_scatter(wv_v, [idx0, jnp.full((LANES,), p, jnp.int32)], vcol * ex)
            return carry
        lax.fori_loop(0, C // LANES, egroup_body, 0)
        pltpu.sync_copy(wv_v, acc_s.at[he_v], add=True)

    start(0, 0)

    def loop2(j, carry):
        i = j * 2
        start(i + 1, 1)
        wait(0)
        compute_scatter(0)
        nxt = jnp.where(i + 2 < NCHUNK, i + 2, 0)
        start(nxt, 0)
        wait(1)
        compute_scatter(1)
        return carry
    lax.fori_loop(0, NCHUNK // 2, loop2, 0)
    wait(0)
    plsc.subcore_barrier()

    def out_acc(o, n):
        pltpu.sync_copy(acc_s.at[pl.ds(rbase + o, n)], wv_v.at[pl.ds(0, n)])
        pltpu.sync_copy(wv_v.at[pl.ds(0, n)], acc_out.at[c, pl.ds(rbase + o, n)])
    _chunked_rows(out_acc)


def kernel(node_feats, hyperedge_index, num_hyperedges, Wk, Wv):
    f32 = jnp.float32
    i32 = jnp.int32

    shift = jnp.asarray(num_hyperedges - H, i32)
    nid = hyperedge_index[0]
    he = hyperedge_index[1] + shift
    pad = EP - E
    nid_p = jnp.concatenate([nid, jnp.zeros((pad,), i32)])
    he_p = jnp.concatenate([he, jnp.full((pad,), H, i32)])
    ztab = jnp.zeros((HP, W), f32)

    katab, kvtab = pl.pallas_call(
        _proj_body,
        out_shape=[jax.ShapeDtypeStruct((N, W), f32),
                   jax.ShapeDtypeStruct((N, W), f32)],
    )(node_feats, Wk, Wv)

    mesh = plsc.VectorSubcoreMesh(core_axis_name="c", subcore_axis_name="s",
                                  num_cores=NC, num_subcores=NS)

    pass_a = pl.kernel(
        _pass_a_body,
        out_type=jax.ShapeDtypeStruct((NC, HP, W), f32),
        mesh=mesh,
        scratch_types=[
            pltpu.VMEM((C,), i32),
            pltpu.VMEM((C,), i32),
            pltpu.VMEM((C, W), f32),
            pltpu.VMEM((C,), i32),
            pltpu.VMEM((C,), i32),
            pltpu.VMEM((C, W), f32),
            pltpu.SemaphoreType.DMA,
            pltpu.SemaphoreType.DMA,
            pltpu.VMEM_SHARED((HP, W), f32),
        ],
    )
    sumk_part = pass_a(nid_p, he_p, katab, ztab)

    ctab = pl.pallas_call(
        _mid_body,
        out_shape=jax.ShapeDtypeStruct((HP, W), f32),
    )(sumk_part)

    pass_b = pl.kernel(
        _pass_b_body,
        out_type=jax.ShapeDtypeStruct((NC, HP, W), f32),
        mesh=mesh,
        compiler_params=pltpu.CompilerParams(needs_layout_passes=False),
        scratch_types=[
            pltpu.VMEM((C,), i32),
            pltpu.VMEM((C,), i32),
            pltpu.VMEM((C, W), f32),
            pltpu.VMEM((C, W), f32),
            pltpu.VMEM((C,), i32),
            pltpu.VMEM((C,), i32),
            pltpu.VMEM((C, W), f32),
            pltpu.VMEM((C, W), f32),
            pltpu.VMEM((C, W), f32),
            pltpu.SemaphoreType.DMA,
            pltpu.SemaphoreType.DMA,
            pltpu.VMEM_SHARED((HP, W), f32),
        ],
    )
    acc_part = pass_b(nid_p, he_p, kvtab, ctab, ztab)

    out = pl.pallas_call(
        _final_body,
        out_shape=jax.ShapeDtypeStruct((H, D), f32),
    )(acc_part, Wv)
    return out
